# baseline (speedup 1.0000x reference)
import functools

import jax
import jax.numpy as jnp
from jax import lax
from jax.experimental import pallas as pl
from jax.experimental.pallas import tpu as pltpu
from jax.experimental.pallas import tpu_sc as plsc

NUM_EDGES = 320000
HIDDEN = 128
EMB_DIM = 48
C0, C1, C2 = 24, 6, 2
NCOMBO = C0 * C1 * C2
CHUNK = 128
NCHUNKS = NUM_EDGES // CHUNK
NC, NS = 2, 16
NW = NC * NS
BASE_T = NCHUNKS // NW
EXTRA = NCHUNKS - BASE_T * NW
NTMAX = BASE_T + 1
SPAN = NTMAX * CHUNK
PAIRS = BASE_T // 2
REP = 16


def _table_body(emb0_ref, emb1_ref, emb2_ref, w_ref, b_ref, t_ref):
    s = lax.broadcasted_iota(jnp.int32, (NCOMBO, 1), 0)
    i0 = s // (C1 * C2)
    i1 = (s // C2) % C1
    i2 = s % C2
    oh0 = (i0 == lax.broadcasted_iota(jnp.int32, (NCOMBO, C0), 1)).astype(jnp.float32)
    oh1 = (i1 == lax.broadcasted_iota(jnp.int32, (NCOMBO, 8), 1)).astype(jnp.float32)
    oh2 = (i2 == lax.broadcasted_iota(jnp.int32, (NCOMBO, 8), 1)).astype(jnp.float32)
    dot = functools.partial(
        jnp.dot, preferred_element_type=jnp.float32, precision=lax.Precision.HIGHEST
    )
    a = dot(oh0, emb0_ref[...]) + dot(oh1, emb1_ref[...]) + dot(oh2, emb2_ref[...])
    h = dot(a, w_ref[...]) + b_ref[...]
    t_ref[...] = 0.5 * h * (1.0 + lax.erf(h * 0.7071067811865476))


_table_call = pl.pallas_call(
    _table_body,
    out_shape=jax.ShapeDtypeStruct((NCOMBO, HIDDEN), jnp.float32),
)


def _make_expand():
    mesh = plsc.VectorSubcoreMesh(core_axis_name="c", subcore_axis_name="s")

    @functools.partial(
        pl.kernel,
        mesh=mesh,
        out_type=jax.ShapeDtypeStruct((NUM_EDGES, HIDDEN), jnp.float32),
        scratch_types=[
            pltpu.VMEM_SHARED((REP * NCOMBO, HIDDEN), jnp.float32),
            pltpu.VMEM((SPAN,), jnp.int32),
            pltpu.VMEM((SPAN,), jnp.int32),
            pltpu.VMEM((SPAN,), jnp.int32),
            pltpu.VMEM((NTMAX, CHUNK), jnp.int32),
            pltpu.VMEM((CHUNK, HIDDEN), jnp.float32),
            pltpu.VMEM((CHUNK, HIDDEN), jnp.float32),
            pltpu.SemaphoreType.DMA,
            pltpu.SemaphoreType.DMA,
            pltpu.SemaphoreType.DMA,
            pltpu.SemaphoreType.DMA,
            pltpu.SemaphoreType.DMA,
        ],
    )
    def expand(
        e0_hbm, e1_hbm, e2_hbm, t_hbm, out_hbm,
        tv, ev0, ev1, ev2, idxs, rows0, rows1, sg0, sg1, sw0, sw1, se,
    ):
        cid = lax.axis_index("c")
        sid = lax.axis_index("s")
        w = sid * NC + cid
        nt = BASE_T + jnp.where(w < EXTRA, 1, 0)
        start_chunk = BASE_T * w + jnp.minimum(w, EXTRA)
        ebase = start_chunk * CHUNK

        @pl.when(sid == 0)
        def _():
            for rep in range(REP):
                pltpu.sync_copy(t_hbm, tv.at[pl.ds(rep * NCOMBO, NCOMBO)])

        c0_ = pltpu.async_copy(e0_hbm.at[pl.ds(ebase, SPAN)], ev0, se)
        c1_ = pltpu.async_copy(e1_hbm.at[pl.ds(ebase, SPAN)], ev1, se)
        c2_ = pltpu.async_copy(e2_hbm.at[pl.ds(ebase, SPAN)], ev2, se)
        c0_.wait()
        c1_.wait()
        c2_.wait()

        lanes = lax.iota(jnp.int32, 16)
        roff = NCOMBO * lax.rem(lanes + w, REP)

        def idx_row(r, carry):
            for k in range(CHUNK // 16):
                sl = pl.ds(r * CHUNK + 16 * k, 16)
                idxs[r, pl.ds(16 * k, 16)] = (
                    (ev0[sl] * C1 + ev1[sl]) * C2 + ev2[sl] + roff
                )
            return carry

        lax.fori_loop(0, nt, idx_row, 0)

        plsc.subcore_barrier()

        def start_gather(r, rowsv, sem):
            return pltpu.async_copy(tv.at[idxs.at[r]], rowsv, sem)

        def wait_gather(r, rowsv, sem):
            pltpu.make_async_copy(tv.at[idxs.at[r]], rowsv, sem).wait()

        def start_write(r, rowsv, sem):
            c = start_chunk + r
            return pltpu.async_copy(rowsv, out_hbm.at[pl.ds(c * CHUNK, CHUNK)], sem)

        def wait_write(r, rowsv, sem):
            c = start_chunk + r
            pltpu.make_async_copy(rowsv, out_hbm.at[pl.ds(c * CHUNK, CHUNK)], sem).wait()

        start_gather(0, rows0, sg0)

        def body(i, carry):
            r0 = 2 * i
            r1 = r0 + 1

            @pl.when(i > 0)
            def _():
                wait_write(r1 - 2, rows1, sw1)

            start_gather(r1, rows1, sg1)
            wait_gather(r0, rows0, sg0)
            start_write(r0, rows0, sw0)

            @pl.when(i < PAIRS - 1)
            def _():
                wait_write(r0, rows0, sw0)
                start_gather(r0 + 2, rows0, sg0)

            wait_gather(r1, rows1, sg1)
            start_write(r1, rows1, sw1)
            return carry

        lax.fori_loop(0, PAIRS, body, 0)

        wait_write(2 * PAIRS - 2, rows0, sw0)
        wait_write(2 * PAIRS - 1, rows1, sw1)

        @pl.when(w < EXTRA)
        def _():
            start_gather(BASE_T, rows0, sg0)
            wait_gather(BASE_T, rows0, sg0)
            pltpu.sync_copy(
                rows0, out_hbm.at[pl.ds((start_chunk + BASE_T) * CHUNK, CHUNK)]
            )

    return expand


_expand_call = _make_expand()


def kernel(e, emb0, emb1, emb2, W, b):
    et = e.T
    pad = SPAN - (NUM_EDGES - (BASE_T * (NW - 1) + EXTRA) * CHUNK)
    e0 = jnp.pad(et[0], (0, pad))
    e1 = jnp.pad(et[1], (0, pad))
    e2 = jnp.pad(et[2], (0, pad))
    emb1p = jnp.pad(emb1, ((0, 8 - C1), (0, 0)))
    emb2p = jnp.pad(emb2, ((0, 8 - C2), (0, 0)))
    table = _table_call(emb0, emb1p, emb2p, W, b.reshape(1, HIDDEN))
    return _expand_call(e0, e1, e2, table)


# speedup vs baseline: 1.1594x; 1.1594x over previous
import functools

import jax
import jax.numpy as jnp
from jax import lax
from jax.experimental import pallas as pl
from jax.experimental.pallas import tpu as pltpu
from jax.experimental.pallas import tpu_sc as plsc

NUM_EDGES = 320000
HIDDEN = 128
EMB_DIM = 48
C0, C1, C2 = 24, 6, 2
NCOMBO = C0 * C1 * C2
CHUNK = 128
BLOCK = 2 * CHUNK
NC, NS = 2, 16
NW = NC * NS
NBLK = 40
SPAN = NBLK * BLOCK
NROW = SPAN // CHUNK


def _table_body(emb0_ref, emb1_ref, emb2_ref, w_ref, b_ref, t_ref):
    s = lax.broadcasted_iota(jnp.int32, (NCOMBO, 1), 0)
    i0 = s // (C1 * C2)
    i1 = (s // C2) % C1
    i2 = s % C2
    oh0 = (i0 == lax.broadcasted_iota(jnp.int32, (NCOMBO, C0), 1)).astype(jnp.float32)
    oh1 = (i1 == lax.broadcasted_iota(jnp.int32, (NCOMBO, 8), 1)).astype(jnp.float32)
    oh2 = (i2 == lax.broadcasted_iota(jnp.int32, (NCOMBO, 8), 1)).astype(jnp.float32)
    dot = functools.partial(
        jnp.dot, preferred_element_type=jnp.float32, precision=lax.Precision.HIGHEST
    )
    a = dot(oh0, emb0_ref[...]) + dot(oh1, emb1_ref[...]) + dot(oh2, emb2_ref[...])
    h = dot(a, w_ref[...]) + b_ref[...]
    t_ref[...] = 0.5 * h * (1.0 + lax.erf(h * 0.7071067811865476))


_table_call = pl.pallas_call(
    _table_body,
    out_shape=jax.ShapeDtypeStruct((NCOMBO, HIDDEN), jnp.float32),
)


def _make_expand():
    mesh = plsc.VectorSubcoreMesh(core_axis_name="c", subcore_axis_name="s")

    @functools.partial(
        pl.kernel,
        mesh=mesh,
        out_type=jax.ShapeDtypeStruct((NUM_EDGES, HIDDEN), jnp.float32),
        scratch_types=[
            pltpu.VMEM_SHARED((NCOMBO, HIDDEN), jnp.float32),
            pltpu.VMEM((SPAN,), jnp.int32),
            pltpu.VMEM((SPAN,), jnp.int32),
            pltpu.VMEM((SPAN,), jnp.int32),
            pltpu.VMEM((NROW, CHUNK), jnp.int32),
            pltpu.VMEM((BLOCK, HIDDEN), jnp.float32),
            pltpu.VMEM((BLOCK, HIDDEN), jnp.float32),
            pltpu.SemaphoreType.DMA,
            pltpu.SemaphoreType.DMA,
            pltpu.SemaphoreType.DMA,
            pltpu.SemaphoreType.DMA,
            pltpu.SemaphoreType.DMA,
        ],
    )
    def expand(
        e0_hbm, e1_hbm, e2_hbm, t_hbm, out_hbm,
        tv, ev0, ev1, ev2, idxs, rows0, rows1, sg0, sg1, sw0, sw1, se,
    ):
        cid = lax.axis_index("c")
        sid = lax.axis_index("s")
        w = sid * NC + cid
        ebase = 8 * ((w * ((NUM_EDGES - SPAN) // 8)) // (NW - 1))

        @pl.when(sid == 0)
        def _():
            pltpu.sync_copy(t_hbm, tv)

        c0_ = pltpu.async_copy(e0_hbm.at[pl.ds(ebase, SPAN)], ev0, se)
        c1_ = pltpu.async_copy(e1_hbm.at[pl.ds(ebase, SPAN)], ev1, se)
        c2_ = pltpu.async_copy(e2_hbm.at[pl.ds(ebase, SPAN)], ev2, se)
        c0_.wait()
        c1_.wait()
        c2_.wait()

        def idx_row(r, carry):
            for k in range(CHUNK // 16):
                sl = pl.ds(r * CHUNK + 16 * k, 16)
                idxs[r, pl.ds(16 * k, 16)] = (ev0[sl] * C1 + ev1[sl]) * C2 + ev2[sl]
            return carry

        lax.fori_loop(0, NROW, idx_row, 0)

        plsc.subcore_barrier()

        def start_gather(j, rowsv, sem):
            pltpu.async_copy(tv.at[idxs.at[2 * j]], rowsv.at[pl.ds(0, CHUNK)], sem)
            pltpu.async_copy(
                tv.at[idxs.at[2 * j + 1]], rowsv.at[pl.ds(CHUNK, CHUNK)], sem
            )

        def wait_gather(j, rowsv, sem):
            pltpu.make_async_copy(
                tv.at[idxs.at[2 * j]], rowsv.at[pl.ds(0, CHUNK)], sem
            ).wait()
            pltpu.make_async_copy(
                tv.at[idxs.at[2 * j + 1]], rowsv.at[pl.ds(CHUNK, CHUNK)], sem
            ).wait()

        def start_write(j, rowsv, sem):
            pltpu.async_copy(
                rowsv, out_hbm.at[pl.ds(ebase + j * BLOCK, BLOCK)], sem
            )

        def wait_write(j, rowsv, sem):
            pltpu.make_async_copy(
                rowsv, out_hbm.at[pl.ds(ebase + j * BLOCK, BLOCK)], sem
            ).wait()

        start_gather(0, rows0, sg0)

        def body(i, carry):
            j0 = 2 * i
            j1 = j0 + 1

            @pl.when(i > 0)
            def _():
                wait_write(j1 - 2, rows1, sw1)

            start_gather(j1, rows1, sg1)
            wait_gather(j0, rows0, sg0)
            start_write(j0, rows0, sw0)

            @pl.when(i < NBLK // 2 - 1)
            def _():
                wait_write(j0, rows0, sw0)
                start_gather(j0 + 2, rows0, sg0)

            wait_gather(j1, rows1, sg1)
            start_write(j1, rows1, sw1)
            return carry

        lax.fori_loop(0, NBLK // 2, body, 0)

        wait_write(NBLK - 2, rows0, sw0)
        wait_write(NBLK - 1, rows1, sw1)

    return expand


_expand_call = _make_expand()


def kernel(e, emb0, emb1, emb2, W, b):
    et = e.T
    emb1p = jnp.pad(emb1, ((0, 8 - C1), (0, 0)))
    emb2p = jnp.pad(emb2, ((0, 8 - C2), (0, 0)))
    table = _table_call(emb0, emb1p, emb2p, W, b.reshape(1, HIDDEN))
    return _expand_call(et[0], et[1], et[2], table)
